# baseline (speedup 1.0000x reference)
import functools

import jax
import jax.numpy as jnp
from jax import lax
from jax.experimental import pallas as pl
from jax.experimental.pallas import tpu as pltpu
from jax.experimental.pallas import tpu_sc as plsc

BATCH = 16384
NROWS = 1000001
DIM = 32


@functools.lru_cache(maxsize=None)
def _build_gather(batch, dim, nrows):
    info = plsc.get_sparse_core_info()
    nw = info.num_cores * info.num_subcores
    bpw = batch // nw
    mesh = plsc.VectorSubcoreMesh(core_axis_name="c", subcore_axis_name="s")

    @functools.partial(
        pl.kernel,
        mesh=mesh,
        out_type=jax.ShapeDtypeStruct((dim, batch), jnp.float32),
        scratch_types=[
            pltpu.VMEM((bpw,), jnp.int32),
            pltpu.VMEM((dim, bpw), jnp.int32),
            pltpu.VMEM((dim, bpw), jnp.float32),
            pltpu.SemaphoreType.DMA,
        ],
        compiler_params=pltpu.CompilerParams(use_tc_tiling_on_sc=False),
    )
    def gather(idx_hbm, wflat_hbm, out_hbm, idx_v, off_v, block_v, sem):
        wid = lax.axis_index("s") * info.num_cores + lax.axis_index("c")
        base = wid * bpw
        pltpu.sync_copy(idx_hbm.at[pl.ds(base, bpw)], idx_v)

        def body(j, _):
            for v in range(bpw // 16):
                sl = pl.ds(v * 16, 16)
                off_v[j, sl] = idx_v[sl] + j * nrows
            pltpu.async_copy(
                wflat_hbm.at[off_v.at[j]], block_v.at[j], sem
            )
            return _

        lax.fori_loop(0, dim, body, 0)

        def drain(j, _):
            pltpu.make_async_copy(
                wflat_hbm.at[off_v.at[j]], block_v.at[j], sem
            ).wait()
            return _

        lax.fori_loop(0, dim, drain, 0)
        pltpu.sync_copy(block_v, out_hbm.at[:, pl.ds(base, bpw)])

    return gather


def kernel(indices, emb_weight):
    wflat = emb_weight.T.reshape(-1)
    out_t = _build_gather(BATCH, DIM, NROWS)(indices.astype(jnp.int32), wflat)
    return out_t.T
